# baseline (speedup 1.0000x reference)
import functools

import jax
import jax.numpy as jnp
from jax.experimental import pallas as pl
from jax.experimental.pallas import tpu as pltpu

EPS_GUARD = 0.0001


def _ffn_body(x_ref, m_ref, w1_ref, b1_ref, w2_ref, b2_ref, o_ref,
              h_ref, *, n_gsteps, n_ftiles):
    g = pl.program_id(1)
    slot = jax.lax.rem(g, 2)

    h = jnp.dot(x_ref[...], w1_ref[0], preferred_element_type=jnp.float32)
    h_ref[slot] = jnp.maximum(h + b1_ref[0], 0.0).astype(jnp.bfloat16)

    part = jnp.dot(h_ref[1 - slot], w2_ref[0],
                   preferred_element_type=jnp.float32)
    cb = ((jax.lax.rem(g - 1, n_ftiles) == 0) & (g > 0)).astype(jnp.float32)
    contrib = jnp.where(g == 0, 0.0, (part + cb * b2_ref[0]) * m_ref[0])
    o_ref[...] = jnp.where(g == 0, 0.0, o_ref[...]) + contrib


def kernel(x, presence, W1, b1, W2, b2):
    N, D = x.shape
    E, _, F = W1.shape

    TN = min(1024, N)
    TF = min(1024, F)
    n_itiles = N // TN
    n_ftiles = F // TF
    n_gsteps = E * n_ftiles

    mask = (presence.T > EPS_GUARD).astype(jnp.float32)[:, :, None]
    b1r = b1[:, None, :]
    b2r = b2[:, None, :]
    xb = x.astype(jnp.bfloat16)
    W1b = W1.astype(jnp.bfloat16)
    W2b = W2.astype(jnp.bfloat16)

    def w1_idx(i, g):
        gc = jnp.minimum(g, n_gsteps - 1)
        return (gc // n_ftiles, 0, jax.lax.rem(gc, n_ftiles))

    def b1_idx(i, g):
        gc = jnp.minimum(g, n_gsteps - 1)
        return (gc // n_ftiles, 0, jax.lax.rem(gc, n_ftiles))

    def w2_idx(i, g):
        gp = jnp.maximum(g - 1, 0)
        return (gp // n_ftiles, jax.lax.rem(gp, n_ftiles), 0)

    def e_prev_idx(i, g):
        return (jnp.maximum(g - 1, 0) // n_ftiles, i, 0)

    def b2_idx(i, g):
        return (jnp.maximum(g - 1, 0) // n_ftiles, 0, 0)

    body = functools.partial(_ffn_body, n_gsteps=n_gsteps, n_ftiles=n_ftiles)

    out = pl.pallas_call(
        body,
        grid=(n_itiles, n_gsteps + 1),
        in_specs=[
            pl.BlockSpec((TN, D), lambda i, g: (i, 0)),
            pl.BlockSpec((1, TN, 1), e_prev_idx),
            pl.BlockSpec((1, D, TF), w1_idx),
            pl.BlockSpec((1, 1, TF), b1_idx),
            pl.BlockSpec((1, TF, D), w2_idx),
            pl.BlockSpec((1, 1, D), b2_idx),
        ],
        out_specs=pl.BlockSpec((TN, D), lambda i, g: (i, 0)),
        out_shape=jax.ShapeDtypeStruct((N, D), jnp.float32),
        scratch_shapes=[
            pltpu.VMEM((2, TN, TF), jnp.bfloat16),
        ],
        compiler_params=pltpu.CompilerParams(
            dimension_semantics=("parallel", "arbitrary"),
        ),
    )(xb, mask, W1b, b1r, W2b, b2r)
    return out


# speedup vs baseline: 1.0869x; 1.0869x over previous
import functools

import jax
import jax.numpy as jnp
from jax.experimental import pallas as pl
from jax.experimental.pallas import tpu as pltpu

EPS_GUARD = 0.0001


def _ffn_body(x_ref, m_ref, w1_ref, b1_ref, w2_ref, b2_ref, o_ref,
              *, n_experts):
    e = pl.program_id(1)

    h = jnp.dot(x_ref[...], w1_ref[0], preferred_element_type=jnp.float32)
    h = jnp.maximum(h + b1_ref[0], 0.0).astype(jnp.bfloat16)
    part = jnp.dot(h, w2_ref[0], preferred_element_type=jnp.float32)
    contrib = (part + b2_ref[0]) * m_ref[0]

    @pl.when(e == 0)
    def _first():
        o_ref[...] = contrib

    @pl.when(e > 0)
    def _rest():
        o_ref[...] += contrib


def kernel(x, presence, W1, b1, W2, b2):
    N, D = x.shape
    E, _, F = W1.shape

    TN = min(1024, N)
    n_itiles = N // TN

    mask = (presence.T > EPS_GUARD).astype(jnp.float32)[:, :, None]
    b1r = b1[:, None, :]
    b2r = b2[:, None, :]
    xb = x.astype(jnp.bfloat16)
    W1b = W1.astype(jnp.bfloat16)
    W2b = W2.astype(jnp.bfloat16)

    body = functools.partial(_ffn_body, n_experts=E)

    out = pl.pallas_call(
        body,
        grid=(n_itiles, E),
        in_specs=[
            pl.BlockSpec((TN, D), lambda i, e: (i, 0)),
            pl.BlockSpec((1, TN, 1), lambda i, e: (e, i, 0)),
            pl.BlockSpec((1, D, F), lambda i, e: (e, 0, 0)),
            pl.BlockSpec((1, 1, F), lambda i, e: (e, 0, 0)),
            pl.BlockSpec((1, F, D), lambda i, e: (e, 0, 0)),
            pl.BlockSpec((1, 1, D), lambda i, e: (e, 0, 0)),
        ],
        out_specs=pl.BlockSpec((TN, D), lambda i, e: (i, 0)),
        out_shape=jax.ShapeDtypeStruct((N, D), jnp.float32),
        compiler_params=pltpu.CompilerParams(
            dimension_semantics=("parallel", "arbitrary"),
        ),
    )(xb, mask, W1b, b1r, W2b, b2r)
    return out
